# baseline (speedup 1.0000x reference)
import jax
import jax.numpy as jnp
from jax import lax
from jax.experimental import pallas as pl
from jax.experimental.pallas import tpu as pltpu
from jax.experimental.pallas import tpu_sc as plsc

NTIME = 2048
NBATCH = 4096
LANES = 16
NWORKERS = 32
W = NBATCH // NWORKERS
NG = W // LANES
CSTRIDE = 8
NC_ROWS = NTIME // CSTRIDE
NPROBE = CSTRIDE // 2


def _interp_body(times_hbm, values_hbm, t_hbm, out_hbm, *scratch):
    t_v, cidx_v, coarse_v, pidx2 = (scratch[0], scratch[1], scratch[2],
                                    scratch[3])
    pidx = [pidx2.at[p] for p in range(NPROBE)]
    pbuf = scratch[4:4 + NPROBE]
    out_v = scratch[4 + NPROBE]
    sem = scratch[5 + NPROBE]

    nc = 2
    wid = lax.axis_index("s") * nc + lax.axis_index("c")
    base = wid * W
    cs = pl.ds(base, W)

    lane = lax.iota(jnp.int32, LANES)

    for j in range(NC_ROWS // LANES):
        cidx_v[pl.ds(j * LANES, LANES)] = (lane + j * LANES) * CSTRIDE + (
            CSTRIDE - 1)
    ct = pltpu.async_copy(times_hbm.at[cidx_v, cs], coarse_v, sem)
    pltpu.sync_copy(t_hbm.at[cs], t_v)
    t_regs = [t_v[pl.ds(g * LANES, LANES)] for g in range(NG)]
    ct.wait()

    loc = [lane + g * LANES for g in range(NG)]
    posc = [jnp.zeros((LANES,), jnp.int32) for _ in range(NG)]
    step = NC_ROWS
    while step >= 1:
        for g in range(NG):
            row = jnp.minimum(posc[g] + (step - 1), NC_ROWS - 1)
            val = plsc.load_gather(coarse_v, [row, loc[g]])
            ok = jnp.logical_and(posc[g] + step <= NC_ROWS,
                                 val <= t_regs[g])
            posc[g] = posc[g] + jnp.where(ok, step, 0)
        step //= 2

    ws = [p * CSTRIDE for p in posc]
    for p in range(NPROBE):
        for g in range(NG):
            pidx[p][pl.ds(g * LANES, LANES)] = jnp.minimum(
                ws[g] + 2 * p, NTIME - 1)
    pcopies = [
        pltpu.async_copy(times_hbm.at[pidx[p], cs], pbuf[p], sem)
        for p in range(NPROBE)
    ]
    for c in pcopies:
        c.wait()

    ms = []
    for g in range(NG):
        m = jnp.zeros((LANES,), jnp.int32)
        for p in range(NPROBE):
            val = plsc.load_gather(pbuf[p], [loc[g], loc[g]])
            m = m + jnp.where(val <= t_regs[g], 1, 0)
        ms.append(m)

    specials = []
    for g in range(NG):
        rbase = ws[g] + 2 * ms[g] - 1
        special = rbase < 0
        specials.append(special)
        r_eff = jnp.where(special, NTIME - 1,
                          jnp.minimum(rbase, NTIME - 1))
        gsl = pl.ds(g * LANES, LANES)
        pidx[0][gsl] = r_eff - 1
        pidx[1][gsl] = r_eff
        pidx[2][gsl] = jnp.minimum(r_eff + 1, NTIME - 1)
    fdst = [pbuf[0], pbuf[1], pbuf[2], pbuf[3],
            coarse_v.at[pl.ds(0, W)], coarse_v.at[pl.ds(W, W)]]
    fcopies = []
    for k in range(3):
        fcopies.append(pltpu.async_copy(times_hbm.at[pidx[k], cs],
                                        fdst[k], sem))
        fcopies.append(pltpu.async_copy(values_hbm.at[pidx[k], cs],
                                        fdst[3 + k], sem))
    for c in fcopies:
        c.wait()

    for g in range(NG):
        t_rm1 = plsc.load_gather(fdst[0], [loc[g], loc[g]])
        t_r = plsc.load_gather(fdst[1], [loc[g], loc[g]])
        t_r1 = plsc.load_gather(fdst[2], [loc[g], loc[g]])
        v_rm1 = plsc.load_gather(fdst[3], [loc[g], loc[g]])
        v_r = plsc.load_gather(coarse_v, [loc[g], loc[g]])
        v_r1 = plsc.load_gather(coarse_v, [W + loc[g], loc[g]])
        ok = t_r <= t_regs[g]
        rbase = ws[g] + 2 * ms[g] - 1
        cnt = jnp.where(
            specials[g], 0,
            jnp.minimum(rbase + jnp.where(ok, 1, 0), NTIME))
        wrap = jnp.logical_or(specials[g], cnt == NTIME)
        use_hi = jnp.logical_and(ok, jnp.logical_not(wrap))
        tk = jnp.where(use_hi, t_r, t_rm1)
        tk1 = jnp.where(use_hi, t_r1, t_r)
        vk = jnp.where(use_hi, v_r, v_rm1)
        vk1 = jnp.where(use_hi, v_r1, v_r)
        s0 = (vk1 - vk) / (tk1 - tk)
        v0 = jnp.where(wrap, vk1, vk)
        t0 = jnp.where(wrap, tk1, tk)
        out_v[pl.ds(g * LANES, LANES)] = v0 + s0 * (t_regs[g] - t0)

    pltpu.sync_copy(out_v, out_hbm.at[cs])


def kernel(times, values, t):
    mesh = plsc.VectorSubcoreMesh(core_axis_name="c", subcore_axis_name="s")
    scratch = [
        pltpu.VMEM((W,), jnp.float32),
        pltpu.VMEM((NC_ROWS,), jnp.int32),
        pltpu.VMEM((NC_ROWS, W), jnp.float32),
        pltpu.VMEM((NPROBE, W), jnp.int32),
    ]
    scratch += [pltpu.VMEM((W, W), jnp.float32) for _ in range(NPROBE)]
    scratch += [
        pltpu.VMEM((W,), jnp.float32),
        pltpu.SemaphoreType.DMA,
    ]
    f = pl.kernel(
        _interp_body,
        mesh=mesh,
        out_type=jax.ShapeDtypeStruct((NBATCH,), jnp.float32),
        compiler_params=pltpu.CompilerParams(needs_layout_passes=False),
        scratch_types=scratch,
    )
    return f(times, values, t)


# speedup vs baseline: 1.0615x; 1.0615x over previous
import jax
import jax.numpy as jnp
from jax import lax
from jax.experimental import pallas as pl
from jax.experimental.pallas import tpu as pltpu
from jax.experimental.pallas import tpu_sc as plsc

NTIME = 2048
NBATCH = 4096
LANES = 16
NWORKERS = 32
W = NBATCH // NWORKERS
NG = W // LANES
CSTRIDE = 16
NC_ROWS = NTIME // CSTRIDE


def _interp_body(times_hbm, values_hbm, t_hbm, out_hbm,
                 t_v, idx_v, idx2_v, coarse_v, fine_v, finv_v, finv1_v,
                 ttail_v, out_v, sem, sem2):
    nc = 2
    wid = lax.axis_index("s") * nc + lax.axis_index("c")
    base = wid * W
    cs = pl.ds(base, W)

    lane = lax.iota(jnp.int32, LANES)

    for g in range(NG):
        idx_v[pl.ds(g * LANES, LANES)] = (lane + g * LANES) * CSTRIDE + (
            CSTRIDE - 1)
    ct = pltpu.async_copy(times_hbm.at[idx_v, cs], coarse_v, sem)
    tt = pltpu.async_copy(times_hbm.at[pl.ds(NTIME - 8, 8), cs], ttail_v,
                          sem2)
    pltpu.sync_copy(t_hbm.at[cs], t_v)
    t_regs = [t_v[pl.ds(g * LANES, LANES)] for g in range(NG)]
    ct.wait()

    loc = [lane + g * LANES for g in range(NG)]
    posc = [jnp.zeros((LANES,), jnp.int32) for _ in range(NG)]
    zero = jnp.zeros((LANES,), jnp.float32)
    tk = [zero for _ in range(NG)]
    tk1 = [zero for _ in range(NG)]
    step = NC_ROWS
    while step >= 1:
        for g in range(NG):
            row = jnp.minimum(posc[g] + (step - 1), NC_ROWS - 1)
            val = plsc.load_gather(coarse_v, [row, loc[g]])
            valid = posc[g] + step <= NC_ROWS
            le = val <= t_regs[g]
            ok = jnp.logical_and(valid, le)
            fail = jnp.logical_and(valid, jnp.logical_not(le))
            tk[g] = jnp.where(ok, val, tk[g])
            tk1[g] = jnp.where(fail, val, tk1[g])
            posc[g] = posc[g] + jnp.where(ok, step, 0)
        step //= 2

    pos = [p * CSTRIDE for p in posc]
    step = CSTRIDE // 2
    while step >= 1:
        for g in range(NG):
            row = jnp.minimum(pos[g] + (step - 1), NTIME - 1)
            idx_v[pl.ds(g * LANES, LANES)] = row
        pltpu.async_copy(times_hbm.at[idx_v, cs], fine_v, sem).wait()
        for g in range(NG):
            val = plsc.load_gather(fine_v, [loc[g], loc[g]])
            valid = pos[g] + step <= NTIME
            le = val <= t_regs[g]
            ok = jnp.logical_and(valid, le)
            fail = jnp.logical_and(valid, jnp.logical_not(le))
            tk[g] = jnp.where(ok, val, tk[g])
            tk1[g] = jnp.where(fail, val, tk1[g])
            pos[g] = pos[g] + jnp.where(ok, step, 0)
        step //= 2

    wraps = []
    for g in range(NG):
        gi = jnp.bitwise_and(pos[g], NTIME - 1)
        wrap = gi == 0
        wraps.append(wrap)
        k = jnp.where(wrap, NTIME - 2, gi - 1)
        idx_v[pl.ds(g * LANES, LANES)] = k
        idx2_v[pl.ds(g * LANES, LANES)] = k + 1
    c0 = pltpu.async_copy(values_hbm.at[idx_v, cs], finv_v, sem)
    c1 = pltpu.async_copy(values_hbm.at[idx2_v, cs], finv1_v, sem)
    tt.wait()
    c0.wait()
    c1.wait()

    for g in range(NG):
        vk = plsc.load_gather(finv_v, [loc[g], loc[g]])
        vk1 = plsc.load_gather(finv1_v, [loc[g], loc[g]])
        gsl = pl.ds(g * LANES, LANES)
        tkf = jnp.where(wraps[g], ttail_v[6, gsl], tk[g])
        tk1f = jnp.where(wraps[g], ttail_v[7, gsl], tk1[g])
        s0 = (vk1 - vk) / (tk1f - tkf)
        v0 = jnp.where(wraps[g], vk1, vk)
        t0 = jnp.where(wraps[g], tk1f, tkf)
        out_v[gsl] = v0 + s0 * (t_regs[g] - t0)

    pltpu.sync_copy(out_v, out_hbm.at[cs])


def kernel(times, values, t):
    mesh = plsc.VectorSubcoreMesh(core_axis_name="c", subcore_axis_name="s")
    f = pl.kernel(
        _interp_body,
        mesh=mesh,
        out_type=jax.ShapeDtypeStruct((NBATCH,), jnp.float32),
        compiler_params=pltpu.CompilerParams(needs_layout_passes=False),
        scratch_types=[
            pltpu.VMEM((W,), jnp.float32),
            pltpu.VMEM((W,), jnp.int32),
            pltpu.VMEM((W,), jnp.int32),
            pltpu.VMEM((NC_ROWS, W), jnp.float32),
            pltpu.VMEM((W, W), jnp.float32),
            pltpu.VMEM((W, W), jnp.float32),
            pltpu.VMEM((W, W), jnp.float32),
            pltpu.VMEM((8, W), jnp.float32),
            pltpu.VMEM((W,), jnp.float32),
            pltpu.SemaphoreType.DMA,
            pltpu.SemaphoreType.DMA,
        ],
    )
    return f(times, values, t)
